# baseline (speedup 1.0000x reference)
import functools

import jax
import jax.numpy as jnp
from jax import lax
from jax.experimental import pallas as pl
from jax.experimental.pallas import tpu as pltpu
from jax.experimental.pallas import tpu_sc as plsc

K = 8192
D = 256
N = 16384
CC = 0.25

R1 = 1024
KT1 = 1024
NI1 = N // R1
NJ1 = K // KT1

R2 = 1024
KT2 = 2048
NI2 = N // R2
NJ2 = K // KT2

R3 = 1024
NI3 = N // R3


CHUNK0_END = 2736
CHUNK1_END = 5472
JB0, SPLIT0 = CHUNK0_END // KT1, CHUNK0_END % KT1
JB1, SPLIT1 = CHUNK1_END // KT1, CHUNK1_END % KT1


def argmin_body(x_ref, w_ref, sx_ref, sw_ref, idx_ref, minv_ref, mini_ref):
    j = pl.program_id(1)
    mm = lax.dot_general(
        x_ref[...], w_ref[...], (((1,), (1,)), ((), ())),
        preferred_element_type=jnp.float32)
    d = (sx_ref[...] + sw_ref[...]) + mm
    lane = lax.broadcasted_iota(jnp.int32, (R1, KT1), 1)
    inf = jnp.float32(jnp.inf)

    def tile_argmin(dm):
        tv = jnp.min(dm, axis=1, keepdims=True)
        cand = jnp.where(dm == tv, lane, KT1)
        tl = jnp.min(cand, axis=1, keepdims=True).astype(jnp.int32)
        return tv, tl + j * KT1

    def merge(tv, ti):
        better = tv < minv_ref[...]
        minv_ref[...] = jnp.where(better, tv, minv_ref[...])
        mini_ref[...] = jnp.where(better, ti, mini_ref[...])

    @pl.when(j == 0)
    def _():
        minv_ref[...] = jnp.full((R1, 1), inf, jnp.float32)
        mini_ref[...] = jnp.zeros((R1, 1), jnp.int32)

    @pl.when(jnp.logical_and(j != JB0, j != JB1))
    def _():
        tv, ti = tile_argmin(d)
        merge(tv, ti)

    @pl.when(j == JB0)
    def _():
        tv, ti = tile_argmin(jnp.where(lane < SPLIT0, d, inf))
        merge(tv, ti)
        minv_ref[...] = minv_ref[...].astype(jnp.bfloat16).astype(jnp.float32)
        tv, ti = tile_argmin(jnp.where(lane >= SPLIT0, d, inf))
        merge(tv, ti)

    @pl.when(j == JB1)
    def _():
        tv, ti = tile_argmin(jnp.where(lane < SPLIT1, d, inf))
        merge(tv, ti)
        minv_ref[...] = minv_ref[...].astype(jnp.bfloat16).astype(jnp.float32)
        tv, ti = tile_argmin(jnp.where(lane >= SPLIT1, d, inf))
        merge(tv, ti)

    @pl.when(j == NJ1 - 1)
    def _():
        idx_ref[...] = mini_ref[...]


def onehot_body(idx_ref, enc_ref, cnt_ref, acc_ref):
    jb = pl.program_id(0)
    ib = pl.program_id(1)
    ids = idx_ref[...]
    lane = lax.broadcasted_iota(jnp.int32, (R2, KT2), 1) + jb * KT2
    oh = (lane == ids).astype(jnp.float32)
    enc_ref[...] = oh
    part = jnp.sum(oh, axis=0, keepdims=True)

    @pl.when(ib == 0)
    def _():
        acc_ref[...] = part

    @pl.when(ib != 0)
    def _():
        acc_ref[...] = acc_ref[...] + part

    @pl.when(ib == NI2 - 1)
    def _():
        cnt_ref[...] = acc_ref[...]


def finish_body(x_ref, q_ref, cnt_ref, qst_ref, loss_ref, perp_ref, acc_ref):
    i = pl.program_id(0)
    xv = x_ref[...]
    qv = q_ref[...].astype(jnp.bfloat16).astype(jnp.float32)
    diff = qv - xv
    qst_ref[...] = xv + diff
    part = jnp.sum(diff * diff)

    @pl.when(i == 0)
    def _():
        acc_ref[0, 0] = part

    @pl.when(i != 0)
    def _():
        acc_ref[0, 0] = acc_ref[0, 0] + part

    @pl.when(i == NI3 - 1)
    def _():
        loss_ref[0, 0] = CC * (acc_ref[0, 0] / (N * D))
        p = cnt_ref[...] * (1.0 / N)
        ent = jnp.sum(p * jnp.log(p + 1e-10))
        pv = jnp.exp(jnp.broadcast_to(-ent, (1, 128)))
        perp_ref[0, 0] = jnp.max(pv)


def sc_gather(table, idx):
    NC, NS = 2, 16
    NW = NC * NS
    BPW = N // NW
    CH = 128
    NCH = BPW // CH
    mesh = plsc.VectorSubcoreMesh(core_axis_name="c", subcore_axis_name="s")

    @functools.partial(
        pl.kernel, mesh=mesh,
        out_type=jax.ShapeDtypeStruct((N, D), jnp.float32),
        scratch_types=[
            pltpu.VMEM((NCH, CH), jnp.int32),
            pltpu.VMEM((CH, D), jnp.float32),
            pltpu.VMEM((CH, D), jnp.float32),
            pltpu.SemaphoreType.DMA,
            pltpu.SemaphoreType.DMA,
        ],
    )
    def k(table_hbm, idx_hbm, out_hbm, idx_v, rows_a, rows_b, sem_a, sem_b):
        wid = lax.axis_index("s") * NC + lax.axis_index("c")
        base = wid * BPW
        for c in range(NCH):
            pltpu.sync_copy(idx_hbm.at[pl.ds(base + c * CH, CH)], idx_v.at[c])
        bufs = ((rows_a, sem_a), (rows_b, sem_b))
        cps = [None] * NCH
        for c in range(NCH):
            buf, sem = bufs[c % 2]
            cps[c] = pltpu.async_copy(table_hbm.at[idx_v.at[c]], buf, sem)
            if c >= 1:
                pb, _ = bufs[(c - 1) % 2]
                cps[c - 1].wait()
                pltpu.sync_copy(pb, out_hbm.at[pl.ds(base + (c - 1) * CH, CH)])
        buf, _ = bufs[(NCH - 1) % 2]
        cps[NCH - 1].wait()
        pltpu.sync_copy(buf, out_hbm.at[pl.ds(base + (NCH - 1) * CH, CH)])

    return k(table, idx)


def kernel(inputs, embedding_weight):
    b, c, h, w = inputs.shape
    x = jnp.transpose(inputs, (0, 2, 3, 1))
    flat = x.reshape(-1, c)
    sx = jnp.sum(flat ** 2, axis=1, keepdims=True)
    sw = jnp.sum(embedding_weight ** 2, axis=1).reshape(1, K)
    flat_bf = flat.astype(jnp.bfloat16)
    wneg2_bf = (-2.0 * embedding_weight).astype(jnp.bfloat16)

    idx = pl.pallas_call(
        argmin_body,
        grid=(NI1, NJ1),
        in_specs=[
            pl.BlockSpec((R1, D), lambda i, j: (i, 0)),
            pl.BlockSpec((KT1, D), lambda i, j: (j, 0)),
            pl.BlockSpec((R1, 1), lambda i, j: (i, 0)),
            pl.BlockSpec((1, KT1), lambda i, j: (0, j)),
        ],
        out_specs=pl.BlockSpec((R1, 1), lambda i, j: (i, 0)),
        out_shape=jax.ShapeDtypeStruct((N, 1), jnp.int32),
        scratch_shapes=[
            pltpu.VMEM((R1, 1), jnp.float32),
            pltpu.VMEM((R1, 1), jnp.int32),
        ],
    )(flat_bf, wneg2_bf, sx, sw)

    encodings, counts = pl.pallas_call(
        onehot_body,
        grid=(NJ2, NI2),
        in_specs=[pl.BlockSpec((R2, 1), lambda j, i: (i, 0))],
        out_specs=[
            pl.BlockSpec((R2, KT2), lambda j, i: (i, j)),
            pl.BlockSpec((1, KT2), lambda j, i: (0, j)),
        ],
        out_shape=[
            jax.ShapeDtypeStruct((N, K), jnp.float32),
            jax.ShapeDtypeStruct((1, K), jnp.float32),
        ],
        scratch_shapes=[pltpu.VMEM((1, KT2), jnp.float32)],
    )(idx)

    q = sc_gather(embedding_weight, idx.reshape(N))

    qst, loss, perp = pl.pallas_call(
        finish_body,
        grid=(NI3,),
        in_specs=[
            pl.BlockSpec((R3, D), lambda i: (i, 0)),
            pl.BlockSpec((R3, D), lambda i: (i, 0)),
            pl.BlockSpec((1, K), lambda i: (0, 0)),
        ],
        out_specs=[
            pl.BlockSpec((R3, D), lambda i: (i, 0)),
            pl.BlockSpec((1, 1), lambda i: (0, 0), memory_space=pltpu.SMEM),
            pl.BlockSpec((1, 1), lambda i: (0, 0), memory_space=pltpu.SMEM),
        ],
        out_shape=[
            jax.ShapeDtypeStruct((N, D), jnp.float32),
            jax.ShapeDtypeStruct((1, 1), jnp.float32),
            jax.ShapeDtypeStruct((1, 1), jnp.float32),
        ],
        scratch_shapes=[pltpu.SMEM((1, 1), jnp.float32)],
    )(flat, q, counts)

    quantized_out = jnp.transpose(qst.reshape(b, h, w, c), (0, 3, 1, 2))
    return (loss[0, 0], quantized_out, perp[0, 0], encodings)


# speedup vs baseline: 1.0319x; 1.0319x over previous
import functools

import jax
import jax.numpy as jnp
from jax import lax
from jax.experimental import pallas as pl
from jax.experimental.pallas import tpu as pltpu
from jax.experimental.pallas import tpu_sc as plsc

K = 8192
D = 256
N = 16384
CC = 0.25

R1 = 1024
KT1 = 1024
NI1 = N // R1
NJ1 = K // KT1

R2 = 2048
KT2 = 2048
NI2 = N // R2
NJ2 = K // KT2

R3 = 1024
NI3 = N // R3


CHUNK0_END = 2736
CHUNK1_END = 5472
JB0, SPLIT0 = CHUNK0_END // KT1, CHUNK0_END % KT1
JB1, SPLIT1 = CHUNK1_END // KT1, CHUNK1_END % KT1


def argmin_body(x_ref, w_ref, sx_ref, sw_ref, idx_ref, minv_ref, mini_ref):
    j = pl.program_id(1)
    mm = lax.dot_general(
        x_ref[...], w_ref[...], (((1,), (1,)), ((), ())),
        preferred_element_type=jnp.float32)
    d = (sx_ref[...] + sw_ref[...]) - 2.0 * mm
    lane = lax.broadcasted_iota(jnp.int32, (R1, KT1), 1)
    inf = jnp.float32(jnp.inf)

    def tile_argmin(dm):
        tv = jnp.min(dm, axis=1, keepdims=True)
        cand = jnp.where(dm == tv, lane, KT1)
        tl = jnp.min(cand, axis=1, keepdims=True).astype(jnp.int32)
        return tv, tl + j * KT1

    def merge(tv, ti):
        better = tv < minv_ref[...]
        minv_ref[...] = jnp.where(better, tv, minv_ref[...])
        mini_ref[...] = jnp.where(better, ti, mini_ref[...])

    @pl.when(j == 0)
    def _():
        minv_ref[...] = jnp.full((R1, 1), inf, jnp.float32)
        mini_ref[...] = jnp.zeros((R1, 1), jnp.int32)

    @pl.when(jnp.logical_and(j != JB0, j != JB1))
    def _():
        tv, ti = tile_argmin(d)
        merge(tv, ti)

    @pl.when(j == JB0)
    def _():
        tv, ti = tile_argmin(jnp.where(lane < SPLIT0, d, inf))
        merge(tv, ti)
        minv_ref[...] = minv_ref[...].astype(jnp.bfloat16).astype(jnp.float32)
        tv, ti = tile_argmin(jnp.where(lane >= SPLIT0, d, inf))
        merge(tv, ti)

    @pl.when(j == JB1)
    def _():
        tv, ti = tile_argmin(jnp.where(lane < SPLIT1, d, inf))
        merge(tv, ti)
        minv_ref[...] = minv_ref[...].astype(jnp.bfloat16).astype(jnp.float32)
        tv, ti = tile_argmin(jnp.where(lane >= SPLIT1, d, inf))
        merge(tv, ti)

    @pl.when(j == NJ1 - 1)
    def _():
        idx_ref[...] = mini_ref[...]


def onehot_body(idx_ref, enc_ref, cnt_ref, acc_ref):
    jb = pl.program_id(0)
    ib = pl.program_id(1)
    ids = idx_ref[...]
    lane = lax.broadcasted_iota(jnp.int32, (R2, KT2), 1) + jb * KT2
    oh = (lane == ids).astype(jnp.float32)
    enc_ref[...] = oh
    part = jnp.sum(oh, axis=0, keepdims=True)

    @pl.when(ib == 0)
    def _():
        acc_ref[...] = part

    @pl.when(ib != 0)
    def _():
        acc_ref[...] = acc_ref[...] + part

    @pl.when(ib == NI2 - 1)
    def _():
        cnt_ref[...] = acc_ref[...]


def finish_body(x_ref, q_ref, cnt_ref, qst_ref, loss_ref, perp_ref, acc_ref):
    i = pl.program_id(0)
    xv = x_ref[...]
    qv = q_ref[...].astype(jnp.bfloat16).astype(jnp.float32)
    diff = qv - xv
    qst_ref[...] = xv + diff
    part = jnp.sum(diff * diff)

    @pl.when(i == 0)
    def _():
        acc_ref[0, 0] = part

    @pl.when(i != 0)
    def _():
        acc_ref[0, 0] = acc_ref[0, 0] + part

    @pl.when(i == NI3 - 1)
    def _():
        loss_ref[0, 0] = CC * (acc_ref[0, 0] / (N * D))
        p = cnt_ref[...] * (1.0 / N)
        ent = jnp.sum(p * jnp.log(p + 1e-10))
        pv = jnp.exp(jnp.broadcast_to(-ent, (1, 128)))
        perp_ref[0, 0] = jnp.max(pv)


def sc_gather(table, idx):
    NC, NS = 2, 16
    NW = NC * NS
    BPW = N // NW
    CH = 128
    NCH = BPW // CH
    mesh = plsc.VectorSubcoreMesh(core_axis_name="c", subcore_axis_name="s")

    @functools.partial(
        pl.kernel, mesh=mesh,
        out_type=jax.ShapeDtypeStruct((N, D), jnp.float32),
        scratch_types=[
            pltpu.VMEM((NCH, CH), jnp.int32),
            pltpu.VMEM((CH, D), jnp.float32),
            pltpu.VMEM((CH, D), jnp.float32),
            pltpu.SemaphoreType.DMA,
            pltpu.SemaphoreType.DMA,
        ],
    )
    def k(table_hbm, idx_hbm, out_hbm, idx_v, rows_a, rows_b, sem_a, sem_b):
        wid = lax.axis_index("s") * NC + lax.axis_index("c")
        base = wid * BPW
        for c in range(NCH):
            pltpu.sync_copy(idx_hbm.at[pl.ds(base + c * CH, CH)], idx_v.at[c])
        bufs = ((rows_a, sem_a), (rows_b, sem_b))
        cps = [None] * NCH
        for c in range(NCH):
            buf, sem = bufs[c % 2]
            cps[c] = pltpu.async_copy(table_hbm.at[idx_v.at[c]], buf, sem)
            if c >= 1:
                pb, _ = bufs[(c - 1) % 2]
                cps[c - 1].wait()
                pltpu.sync_copy(pb, out_hbm.at[pl.ds(base + (c - 1) * CH, CH)])
        buf, _ = bufs[(NCH - 1) % 2]
        cps[NCH - 1].wait()
        pltpu.sync_copy(buf, out_hbm.at[pl.ds(base + (NCH - 1) * CH, CH)])

    return k(table, idx)


def kernel(inputs, embedding_weight):
    b, c, h, w = inputs.shape
    x = jnp.transpose(inputs, (0, 2, 3, 1))
    flat = x.reshape(-1, c)
    sx = jnp.sum(flat ** 2, axis=1, keepdims=True)
    sw = jnp.sum(embedding_weight ** 2, axis=1).reshape(1, K)

    idx = pl.pallas_call(
        argmin_body,
        grid=(NI1, NJ1),
        in_specs=[
            pl.BlockSpec((R1, D), lambda i, j: (i, 0)),
            pl.BlockSpec((KT1, D), lambda i, j: (j, 0)),
            pl.BlockSpec((R1, 1), lambda i, j: (i, 0)),
            pl.BlockSpec((1, KT1), lambda i, j: (0, j)),
        ],
        out_specs=pl.BlockSpec((R1, 1), lambda i, j: (i, 0)),
        out_shape=jax.ShapeDtypeStruct((N, 1), jnp.int32),
        scratch_shapes=[
            pltpu.VMEM((R1, 1), jnp.float32),
            pltpu.VMEM((R1, 1), jnp.int32),
        ],
    )(flat, embedding_weight, sx, sw)

    q = sc_gather(embedding_weight, idx.reshape(N))

    encodings, counts = pl.pallas_call(
        onehot_body,
        grid=(NJ2, NI2),
        in_specs=[pl.BlockSpec((R2, 1), lambda j, i: (i, 0))],
        out_specs=[
            pl.BlockSpec((R2, KT2), lambda j, i: (i, j)),
            pl.BlockSpec((1, KT2), lambda j, i: (0, j)),
        ],
        out_shape=[
            jax.ShapeDtypeStruct((N, K), jnp.float32),
            jax.ShapeDtypeStruct((1, K), jnp.float32),
        ],
        scratch_shapes=[pltpu.VMEM((1, KT2), jnp.float32)],
    )(idx)

    qst, loss, perp = pl.pallas_call(
        finish_body,
        grid=(NI3,),
        in_specs=[
            pl.BlockSpec((R3, D), lambda i: (i, 0)),
            pl.BlockSpec((R3, D), lambda i: (i, 0)),
            pl.BlockSpec((1, K), lambda i: (0, 0)),
        ],
        out_specs=[
            pl.BlockSpec((R3, D), lambda i: (i, 0)),
            pl.BlockSpec((1, 1), lambda i: (0, 0), memory_space=pltpu.SMEM),
            pl.BlockSpec((1, 1), lambda i: (0, 0), memory_space=pltpu.SMEM),
        ],
        out_shape=[
            jax.ShapeDtypeStruct((N, D), jnp.float32),
            jax.ShapeDtypeStruct((1, 1), jnp.float32),
            jax.ShapeDtypeStruct((1, 1), jnp.float32),
        ],
        scratch_shapes=[pltpu.SMEM((1, 1), jnp.float32)],
    )(flat, q, counts)

    quantized_out = jnp.transpose(qst.reshape(b, h, w, c), (0, 3, 1, 2))
    return (loss[0, 0], quantized_out, perp[0, 0], encodings)


# speedup vs baseline: 1.0463x; 1.0140x over previous
import functools

import jax
import jax.numpy as jnp
from jax import lax
from jax.experimental import pallas as pl
from jax.experimental.pallas import tpu as pltpu
from jax.experimental.pallas import tpu_sc as plsc

K = 8192
D = 256
N = 16384
CC = 0.25

R1 = 1024
KT1 = 1024
NI1 = N // R1
NJ1 = K // KT1

R2 = 2048
KT2 = 2048
NI2 = N // R2
NJ2 = K // KT2

R3 = 1024
NI3 = N // R3


CHUNK0_END = 2736
CHUNK1_END = 5472
JB0, SPLIT0 = CHUNK0_END // KT1, CHUNK0_END % KT1
JB1, SPLIT1 = CHUNK1_END // KT1, CHUNK1_END % KT1


def argmin_body(x_ref, w_ref, sx_ref, sw_ref, idx_ref, minv_ref, mini_ref):
    j = pl.program_id(1)
    mm = lax.dot_general(
        x_ref[...], w_ref[...], (((1,), (1,)), ((), ())),
        preferred_element_type=jnp.float32)
    d = (sx_ref[...] + sw_ref[...]) - 2.0 * mm
    lane = lax.broadcasted_iota(jnp.int32, (R1, KT1), 1)
    inf = jnp.float32(jnp.inf)

    def tile_argmin(dm):
        tv = jnp.min(dm, axis=1, keepdims=True)
        cand = jnp.where(dm == tv, lane, KT1)
        tl = jnp.min(cand, axis=1, keepdims=True).astype(jnp.int32)
        return tv, tl + j * KT1

    def merge(tv, ti):
        better = tv < minv_ref[...]
        minv_ref[...] = jnp.where(better, tv, minv_ref[...])
        mini_ref[...] = jnp.where(better, ti, mini_ref[...])

    @pl.when(j == 0)
    def _():
        minv_ref[...] = jnp.full((R1, 1), inf, jnp.float32)
        mini_ref[...] = jnp.zeros((R1, 1), jnp.int32)

    @pl.when(jnp.logical_and(j != JB0, j != JB1))
    def _():
        tv, ti = tile_argmin(d)
        merge(tv, ti)

    @pl.when(j == JB0)
    def _():
        tv, ti = tile_argmin(jnp.where(lane < SPLIT0, d, inf))
        merge(tv, ti)
        minv_ref[...] = minv_ref[...].astype(jnp.bfloat16).astype(jnp.float32)
        tv, ti = tile_argmin(jnp.where(lane >= SPLIT0, d, inf))
        merge(tv, ti)

    @pl.when(j == JB1)
    def _():
        tv, ti = tile_argmin(jnp.where(lane < SPLIT1, d, inf))
        merge(tv, ti)
        minv_ref[...] = minv_ref[...].astype(jnp.bfloat16).astype(jnp.float32)
        tv, ti = tile_argmin(jnp.where(lane >= SPLIT1, d, inf))
        merge(tv, ti)

    @pl.when(j == NJ1 - 1)
    def _():
        idx_ref[...] = mini_ref[...]


def onehot_body(idx_ref, x_ref, q_ref, enc_ref, qst_ref, loss_ref, perp_ref,
                cnt_ref, lacc_ref):
    ib = pl.program_id(0)
    jb = pl.program_id(1)
    ids = idx_ref[...]
    lane = lax.broadcasted_iota(jnp.int32, (R2, KT2), 1) + jb * KT2
    oh = (lane == ids).astype(jnp.float32)
    enc_ref[...] = oh
    part = jnp.sum(oh, axis=0, keepdims=True)

    for c in range(NJ2):
        @pl.when(jnp.logical_and(jb == c, ib == 0))
        def _():
            cnt_ref[0:1, c * KT2:(c + 1) * KT2] = part

        @pl.when(jnp.logical_and(jb == c, ib != 0))
        def _():
            cnt_ref[0:1, c * KT2:(c + 1) * KT2] = (
                cnt_ref[0:1, c * KT2:(c + 1) * KT2] + part)

    @pl.when(jb == 0)
    def _():
        xv = x_ref[...]
        qv = q_ref[...].astype(jnp.bfloat16).astype(jnp.float32)
        diff = qv - xv
        qst_ref[...] = xv + diff
        lpart = jnp.sum(diff * diff)

        @pl.when(ib == 0)
        def _():
            lacc_ref[0, 0] = lpart

        @pl.when(ib != 0)
        def _():
            lacc_ref[0, 0] = lacc_ref[0, 0] + lpart

    @pl.when(jnp.logical_and(ib == NI2 - 1, jb == NJ2 - 1))
    def _():
        loss_ref[0, 0] = CC * (lacc_ref[0, 0] / (N * D))
        p = cnt_ref[...] * (1.0 / N)
        ent = jnp.sum(p * jnp.log(p + 1e-10))
        pv = jnp.exp(jnp.broadcast_to(-ent, (1, 128)))
        perp_ref[0, 0] = jnp.max(pv)


def sc_gather(table, idx):
    NC, NS = 2, 16
    NW = NC * NS
    BPW = N // NW
    CH = 128
    NCH = BPW // CH
    mesh = plsc.VectorSubcoreMesh(core_axis_name="c", subcore_axis_name="s")

    @functools.partial(
        pl.kernel, mesh=mesh,
        out_type=jax.ShapeDtypeStruct((N, D), jnp.float32),
        scratch_types=[
            pltpu.VMEM((NCH, CH), jnp.int32),
            pltpu.VMEM((CH, D), jnp.float32),
            pltpu.VMEM((CH, D), jnp.float32),
            pltpu.SemaphoreType.DMA,
            pltpu.SemaphoreType.DMA,
        ],
    )
    def k(table_hbm, idx_hbm, out_hbm, idx_v, rows_a, rows_b, sem_a, sem_b):
        wid = lax.axis_index("s") * NC + lax.axis_index("c")
        base = wid * BPW
        for c in range(NCH):
            pltpu.sync_copy(idx_hbm.at[pl.ds(base + c * CH, CH)], idx_v.at[c])
        bufs = ((rows_a, sem_a), (rows_b, sem_b))
        cps = [None] * NCH
        for c in range(NCH):
            buf, sem = bufs[c % 2]
            cps[c] = pltpu.async_copy(table_hbm.at[idx_v.at[c]], buf, sem)
            if c >= 1:
                pb, _ = bufs[(c - 1) % 2]
                cps[c - 1].wait()
                pltpu.sync_copy(pb, out_hbm.at[pl.ds(base + (c - 1) * CH, CH)])
        buf, _ = bufs[(NCH - 1) % 2]
        cps[NCH - 1].wait()
        pltpu.sync_copy(buf, out_hbm.at[pl.ds(base + (NCH - 1) * CH, CH)])

    return k(table, idx)


def kernel(inputs, embedding_weight):
    b, c, h, w = inputs.shape
    x = jnp.transpose(inputs, (0, 2, 3, 1))
    flat = x.reshape(-1, c)
    sx = jnp.sum(flat ** 2, axis=1, keepdims=True)
    sw = jnp.sum(embedding_weight ** 2, axis=1).reshape(1, K)

    idx = pl.pallas_call(
        argmin_body,
        grid=(NI1, NJ1),
        in_specs=[
            pl.BlockSpec((R1, D), lambda i, j: (i, 0)),
            pl.BlockSpec((KT1, D), lambda i, j: (j, 0)),
            pl.BlockSpec((R1, 1), lambda i, j: (i, 0)),
            pl.BlockSpec((1, KT1), lambda i, j: (0, j)),
        ],
        out_specs=pl.BlockSpec((R1, 1), lambda i, j: (i, 0)),
        out_shape=jax.ShapeDtypeStruct((N, 1), jnp.int32),
        scratch_shapes=[
            pltpu.VMEM((R1, 1), jnp.float32),
            pltpu.VMEM((R1, 1), jnp.int32),
        ],
    )(flat, embedding_weight, sx, sw)

    q = sc_gather(embedding_weight, idx.reshape(N))

    encodings, qst, loss, perp = pl.pallas_call(
        onehot_body,
        grid=(NI2, NJ2),
        in_specs=[
            pl.BlockSpec((R2, 1), lambda i, j: (i, 0)),
            pl.BlockSpec((R2, D), lambda i, j: (i, 0)),
            pl.BlockSpec((R2, D), lambda i, j: (i, 0)),
        ],
        out_specs=[
            pl.BlockSpec((R2, KT2), lambda i, j: (i, j)),
            pl.BlockSpec((R2, D), lambda i, j: (i, 0)),
            pl.BlockSpec((1, 1), lambda i, j: (0, 0), memory_space=pltpu.SMEM),
            pl.BlockSpec((1, 1), lambda i, j: (0, 0), memory_space=pltpu.SMEM),
        ],
        out_shape=[
            jax.ShapeDtypeStruct((N, K), jnp.float32),
            jax.ShapeDtypeStruct((N, D), jnp.float32),
            jax.ShapeDtypeStruct((1, 1), jnp.float32),
            jax.ShapeDtypeStruct((1, 1), jnp.float32),
        ],
        scratch_shapes=[
            pltpu.VMEM((1, K), jnp.float32),
            pltpu.SMEM((1, 1), jnp.float32),
        ],
    )(idx, flat, q)

    quantized_out = jnp.transpose(qst.reshape(b, h, w, c), (0, 3, 1, 2))
    return (loss[0, 0], quantized_out, perp[0, 0], encodings)
